# Initial kernel scaffold; baseline (speedup 1.0000x reference)
import jax
import jax.numpy as jnp
from jax.experimental import pallas as pl


def kernel(x, edge_index, W1_0, W2_0, Wr_0, br_0, W1_1, W2_1, Wr_1, br_1, Wf, bf):
    raise NotImplementedError("write your pallas kernel here")


# speedup vs baseline: 6.0065x; 6.0065x over previous
import functools

import jax
import jax.numpy as jnp
from jax import lax
from jax.experimental import pallas as pl
from jax.experimental.pallas import tpu as pltpu
from jax.experimental.pallas import tpu_sc as plsc

_N = 10000
_E = 320000
_D = 128

_NS = 16
_CHUNK = 125
_EPT = _E // _NS
_NCH = _EPT // _CHUNK
_BS = 16
_NB = _NCH // _BS
_NP = 10240
_RPT = _NP // _NS
_RZ = 128
_RCH = _RPT // _RZ
_CW = 16

_BN = 1000


@functools.lru_cache(maxsize=None)
def _build_seg_kernel():
    mesh = plsc.VectorSubcoreMesh(core_axis_name="c", subcore_axis_name="s")

    @functools.partial(
        pl.kernel,
        out_type=[jax.ShapeDtypeStruct((_NP, _D), jnp.float32),
                  jax.ShapeDtypeStruct((_NP, _D), jnp.float32)],
        mesh=mesh,
        scratch_types=[
            pltpu.VMEM_SHARED((_NP, _D), jnp.float32),
            pltpu.VMEM((_BS, _CHUNK), jnp.int32),
            pltpu.VMEM((_BS, _CHUNK), jnp.int32),
            pltpu.VMEM((_RZ, _D), jnp.float32),
            pltpu.SemaphoreType.DMA,
        ],
    )
    def seg(t1, t2, src2d, dst2d, z128,
            s1_out, s2_out, acc, gidx, sidx, rowbuf, sem):
        cid = lax.axis_index("c")
        tid = lax.axis_index("s")

        def one_pass(table, g2, s3, out):
            pltpu.sync_copy(z128, rowbuf)
            for z in range(_RCH):
                pltpu.sync_copy(
                    rowbuf, acc.at[pl.ds(tid * _RPT + z * _RZ, _RZ)])
            plsc.subcore_barrier()

            def batch(b, carry):
                r = pl.multiple_of(tid * _NCH + b * _BS, _BS)
                pltpu.sync_copy(g2.at[pl.ds(r, _BS)], gidx)
                pltpu.sync_copy(s3.at[pl.ds(r, _BS)], sidx)
                for jj in range(_BS):
                    pltpu.async_copy(table.at[gidx.at[jj]],
                                     rowbuf.at[pl.ds(0, _CHUNK)], sem).wait()
                    pltpu.sync_copy(rowbuf.at[pl.ds(0, _CHUNK)],
                                    acc.at[sidx.at[jj]], add=True)
                return carry

            lax.fori_loop(0, _NB, batch, 0)
            plsc.subcore_barrier()
            for z in range(_RCH):
                r0 = tid * _RPT + z * _RZ
                pltpu.sync_copy(acc.at[pl.ds(r0, _RZ)],
                                out.at[pl.ds(r0, _RZ)])

        @pl.when(cid == 0)
        def _():
            one_pass(t1, src2d, dst2d, s1_out)

        @pl.when(cid == 1)
        def _():
            one_pass(t2, dst2d, src2d, s2_out)

    return seg


@functools.lru_cache(maxsize=None)
def _build_cnt_kernel():
    mesh = plsc.VectorSubcoreMesh(core_axis_name="c", subcore_axis_name="s")

    @functools.partial(
        pl.kernel,
        out_type=[jax.ShapeDtypeStruct((_NP, _D), jnp.float32),
                  jax.ShapeDtypeStruct((_NP, _D), jnp.float32)],
        mesh=mesh,
        scratch_types=[
            pltpu.VMEM_SHARED((_NP, _D), jnp.float32),
            pltpu.VMEM((_BS, _CHUNK), jnp.int32),
            pltpu.VMEM((_CHUNK, _D), jnp.float32),
            pltpu.VMEM((_RZ, _D), jnp.float32),
        ],
    )
    def cnt(src2d, dst2d, ones128, z128,
            cin_out, cout_out, acc, sidx, onesbuf, rowbuf):
        cid = lax.axis_index("c")
        tid = lax.axis_index("s")

        def one_pass(s3, out):
            pltpu.sync_copy(z128, rowbuf)
            for z in range(_RCH):
                pltpu.sync_copy(
                    rowbuf, acc.at[pl.ds(tid * _RPT + z * _RZ, _RZ)])
            pltpu.sync_copy(ones128, onesbuf)
            plsc.subcore_barrier()

            def batch(b, carry):
                r = pl.multiple_of(tid * _NCH + b * _BS, _BS)
                pltpu.sync_copy(s3.at[pl.ds(r, _BS)], sidx)
                for jj in range(_BS):
                    pltpu.sync_copy(onesbuf, acc.at[sidx.at[jj]], add=True)
                return carry

            lax.fori_loop(0, _NB, batch, 0)
            plsc.subcore_barrier()
            for z in range(_RCH):
                r0 = tid * _RPT + z * _RZ
                pltpu.sync_copy(acc.at[pl.ds(r0, _RZ)],
                                out.at[pl.ds(r0, _RZ)])

        @pl.when(cid == 0)
        def _():
            one_pass(dst2d, cin_out)

        @pl.when(cid == 1)
        def _():
            one_pass(src2d, cout_out)

    return cnt


def _dot(a, b):
    return jnp.dot(a, b, preferred_element_type=jnp.float32)


def _mm3_body(x_ref, w1_ref, w2_ref, wr_ref, br_ref, t1_ref, t2_ref, r_ref):
    xb = x_ref[...]
    t1_ref[...] = _dot(xb, w1_ref[...])
    t2_ref[...] = _dot(xb, w2_ref[...])
    r_ref[...] = _dot(xb, wr_ref[...]) + br_ref[...]


def _mm3(x, w1, w2, wr, br):
    nd = jax.ShapeDtypeStruct((_N, _D), jnp.float32)
    row = pl.BlockSpec((_BN, _D), lambda i: (i, 0))
    full = pl.BlockSpec((_D, _D), lambda i: (0, 0))
    bias = pl.BlockSpec((1, _D), lambda i: (0, 0))
    return pl.pallas_call(
        _mm3_body,
        grid=(_N // _BN,),
        in_specs=[row, full, full, full, bias],
        out_specs=[row, row, row],
        out_shape=[nd, nd, nd],
    )(x, w1, w2, wr, br.reshape(1, _D))


def _finish(r_ref, s1_ref, s2_ref, cin_ref, cout_ref):
    rin = 1.0 / jnp.maximum(cin_ref[...], 1.0)
    rout = 1.0 / jnp.maximum(cout_ref[...], 1.0)
    return jnp.maximum(
        r_ref[...] + s1_ref[...] * rin + s2_ref[...] * rout, 0.0)


def _mid_body(r_ref, s1_ref, s2_ref, cin_ref, cout_ref,
              w1_ref, w2_ref, wr_ref, br_ref,
              h_ref, t1_ref, t2_ref, r1_ref):
    h = _finish(r_ref, s1_ref, s2_ref, cin_ref, cout_ref)
    h_ref[...] = h
    t1_ref[...] = _dot(h, w1_ref[...])
    t2_ref[...] = _dot(h, w2_ref[...])
    r1_ref[...] = _dot(h, wr_ref[...]) + br_ref[...]


def _mid(r0, s1, s2, cin, cout, w1, w2, wr, br):
    nd = jax.ShapeDtypeStruct((_N, _D), jnp.float32)
    row = pl.BlockSpec((_BN, _D), lambda i: (i, 0))
    full = pl.BlockSpec((_D, _D), lambda i: (0, 0))
    bias = pl.BlockSpec((1, _D), lambda i: (0, 0))
    return pl.pallas_call(
        _mid_body,
        grid=(_N // _BN,),
        in_specs=[row, row, row, row, row, full, full, full, bias],
        out_specs=[row, row, row, row],
        out_shape=[nd, nd, nd, nd],
    )(r0, s1, s2, cin, cout, w1, w2, wr, br.reshape(1, _D))


def _fin_body(r_ref, s1_ref, s2_ref, cin_ref, cout_ref,
              x_ref, h1_ref, wfa_ref, wfb_ref, wfc_ref, bf_ref, y_ref):
    h2 = _finish(r_ref, s1_ref, s2_ref, cin_ref, cout_ref)
    y_ref[...] = (_dot(x_ref[...], wfa_ref[...])
                  + _dot(h1_ref[...], wfb_ref[...])
                  + _dot(h2, wfc_ref[...]) + bf_ref[...])


def _fin(r1, s1, s2, cin, cout, x, h1, wfa, wfb, wfc, bf):
    nd = jax.ShapeDtypeStruct((_N, _D), jnp.float32)
    row = pl.BlockSpec((_BN, _D), lambda i: (i, 0))
    full = pl.BlockSpec((_D, _D), lambda i: (0, 0))
    bias = pl.BlockSpec((1, _D), lambda i: (0, 0))
    return pl.pallas_call(
        _fin_body,
        grid=(_N // _BN,),
        in_specs=[row, row, row, row, row, row, row, full, full, full, bias],
        out_specs=row,
        out_shape=nd,
    )(r1, s1, s2, cin, cout, x, h1, wfa, wfb, wfc, bf.reshape(1, _D))


def kernel(x, edge_index, W1_0, W2_0, Wr_0, br_0,
           W1_1, W2_1, Wr_1, br_1, Wf, bf):
    src = edge_index[0]
    dst = edge_index[1]
    src2d = src.reshape(_E // _CHUNK, _CHUNK)
    dst2d = dst.reshape(_E // _CHUNK, _CHUNK)
    z128 = jnp.zeros((_RZ, _D), jnp.float32)
    ones128 = jnp.ones((_CHUNK, _D), jnp.float32)

    cnt_k = _build_cnt_kernel()
    seg_k = _build_seg_kernel()

    cin, cout = cnt_k(src2d, dst2d, ones128, z128)

    t1_0, t2_0, r0 = _mm3(x, W1_0, W2_0, Wr_0, br_0)
    s1_0, s2_0 = seg_k(t1_0, t2_0, src2d, dst2d, z128)

    h1, t1_1, t2_1, r1 = _mid(r0, s1_0, s2_0, cin, cout,
                              W1_1, W2_1, Wr_1, br_1)
    s1_1, s2_1 = seg_k(t1_1, t2_1, src2d, dst2d, z128)

    return _fin(r1, s1_1, s2_1, cin, cout, x, h1,
                Wf[:_D], Wf[_D:2 * _D], Wf[2 * _D:], bf)


# speedup vs baseline: 8.3181x; 1.3849x over previous
import functools

import jax
import jax.numpy as jnp
from jax import lax
from jax.experimental import pallas as pl
from jax.experimental.pallas import tpu as pltpu
from jax.experimental.pallas import tpu_sc as plsc

_N = 10000
_E = 320000
_D = 128

_NS = 16
_CHUNK = 125
_EPT = _E // _NS
_NCH = _EPT // _CHUNK
_BS = 16
_NB = _NCH // _BS
_NP = 10240
_RPT = _NP // _NS
_RZ = 128
_RCH = _RPT // _RZ
_CW = 16

_BN = 1000


@functools.lru_cache(maxsize=None)
def _build_seg_kernel():
    mesh = plsc.VectorSubcoreMesh(core_axis_name="c", subcore_axis_name="s")

    @functools.partial(
        pl.kernel,
        out_type=[jax.ShapeDtypeStruct((_NP, _D), jnp.float32),
                  jax.ShapeDtypeStruct((_NP, _D), jnp.float32)],
        mesh=mesh,
        scratch_types=[
            pltpu.VMEM_SHARED((_NP, _D), jnp.float32),
            pltpu.VMEM((_BS, _CHUNK), jnp.int32),
            pltpu.VMEM((_BS, _CHUNK), jnp.int32),
            pltpu.VMEM((_RZ, _D), jnp.float32),
            pltpu.VMEM((_RZ, _D), jnp.float32),
            pltpu.SemaphoreType.DMA,
            pltpu.SemaphoreType.DMA,
        ],
    )
    def seg(t1, t2, src2d, dst2d, z128,
            s1_out, s2_out, acc, gidx, sidx, rowbuf, rowbuf1, sem, sem1):
        cid = lax.axis_index("c")
        tid = lax.axis_index("s")

        def one_pass(table, g2, s3, out):
            pltpu.sync_copy(z128, rowbuf)
            for z in range(_RCH):
                pltpu.sync_copy(
                    rowbuf, acc.at[pl.ds(tid * _RPT + z * _RZ, _RZ)])
            plsc.subcore_barrier()

            bufs = (rowbuf.at[pl.ds(0, _CHUNK)], rowbuf1.at[pl.ds(0, _CHUNK)])
            sems = (sem, sem1)

            def batch(b, carry):
                r = pl.multiple_of(tid * _NCH + b * _BS, _BS)
                pltpu.sync_copy(g2.at[pl.ds(r, _BS)], gidx)
                pltpu.sync_copy(s3.at[pl.ds(r, _BS)], sidx)
                descs = [None, None]
                descs[0] = pltpu.async_copy(table.at[gidx.at[0]], bufs[0],
                                            sems[0])
                for jj in range(_BS):
                    if jj + 1 < _BS:
                        nb = (jj + 1) % 2
                        descs[nb] = pltpu.async_copy(
                            table.at[gidx.at[jj + 1]], bufs[nb], sems[nb])
                    descs[jj % 2].wait()
                    pltpu.sync_copy(bufs[jj % 2], acc.at[sidx.at[jj]],
                                    add=True)
                return carry

            lax.fori_loop(0, _NB, batch, 0)
            plsc.subcore_barrier()
            for z in range(_RCH):
                r0 = tid * _RPT + z * _RZ
                pltpu.sync_copy(acc.at[pl.ds(r0, _RZ)],
                                out.at[pl.ds(r0, _RZ)])

        @pl.when(cid == 0)
        def _():
            one_pass(t1, src2d, dst2d, s1_out)

        @pl.when(cid == 1)
        def _():
            one_pass(t2, dst2d, src2d, s2_out)

    return seg


@functools.lru_cache(maxsize=None)
def _build_cnt_kernel():
    mesh = plsc.VectorSubcoreMesh(core_axis_name="c", subcore_axis_name="s")

    @functools.partial(
        pl.kernel,
        out_type=[jax.ShapeDtypeStruct((_NP, _D), jnp.float32),
                  jax.ShapeDtypeStruct((_NP, _D), jnp.float32)],
        mesh=mesh,
        scratch_types=[
            pltpu.VMEM_SHARED((_NP, _D), jnp.float32),
            pltpu.VMEM((_BS, _CHUNK), jnp.int32),
            pltpu.VMEM((_CHUNK, _D), jnp.float32),
            pltpu.VMEM((_RZ, _D), jnp.float32),
        ],
    )
    def cnt(src2d, dst2d, ones128, z128,
            cin_out, cout_out, acc, sidx, onesbuf, rowbuf):
        cid = lax.axis_index("c")
        tid = lax.axis_index("s")

        def one_pass(s3, out):
            pltpu.sync_copy(z128, rowbuf)
            for z in range(_RCH):
                pltpu.sync_copy(
                    rowbuf, acc.at[pl.ds(tid * _RPT + z * _RZ, _RZ)])
            pltpu.sync_copy(ones128, onesbuf)
            plsc.subcore_barrier()

            def batch(b, carry):
                r = pl.multiple_of(tid * _NCH + b * _BS, _BS)
                pltpu.sync_copy(s3.at[pl.ds(r, _BS)], sidx)
                for jj in range(_BS):
                    pltpu.sync_copy(onesbuf, acc.at[sidx.at[jj]], add=True)
                return carry

            lax.fori_loop(0, _NB, batch, 0)
            plsc.subcore_barrier()
            for z in range(_RCH):
                r0 = tid * _RPT + z * _RZ
                pltpu.sync_copy(acc.at[pl.ds(r0, _RZ)],
                                out.at[pl.ds(r0, _RZ)])

        @pl.when(cid == 0)
        def _():
            one_pass(dst2d, cin_out)

        @pl.when(cid == 1)
        def _():
            one_pass(src2d, cout_out)

    return cnt


def _dot(a, b):
    return jnp.dot(a, b, preferred_element_type=jnp.float32)


def _mm3_body(x_ref, w1_ref, w2_ref, wr_ref, br_ref, t1_ref, t2_ref, r_ref):
    xb = x_ref[...]
    t1_ref[...] = _dot(xb, w1_ref[...])
    t2_ref[...] = _dot(xb, w2_ref[...])
    r_ref[...] = _dot(xb, wr_ref[...]) + br_ref[...]


def _mm3(x, w1, w2, wr, br):
    nd = jax.ShapeDtypeStruct((_N, _D), jnp.float32)
    row = pl.BlockSpec((_BN, _D), lambda i: (i, 0))
    full = pl.BlockSpec((_D, _D), lambda i: (0, 0))
    bias = pl.BlockSpec((1, _D), lambda i: (0, 0))
    return pl.pallas_call(
        _mm3_body,
        grid=(_N // _BN,),
        in_specs=[row, full, full, full, bias],
        out_specs=[row, row, row],
        out_shape=[nd, nd, nd],
    )(x, w1, w2, wr, br.reshape(1, _D))


def _finish(r_ref, s1_ref, s2_ref, cin_ref, cout_ref):
    rin = 1.0 / jnp.maximum(cin_ref[...], 1.0)
    rout = 1.0 / jnp.maximum(cout_ref[...], 1.0)
    return jnp.maximum(
        r_ref[...] + s1_ref[...] * rin + s2_ref[...] * rout, 0.0)


def _mid_body(r_ref, s1_ref, s2_ref, cin_ref, cout_ref,
              w1_ref, w2_ref, wr_ref, br_ref,
              h_ref, t1_ref, t2_ref, r1_ref):
    h = _finish(r_ref, s1_ref, s2_ref, cin_ref, cout_ref)
    h_ref[...] = h
    t1_ref[...] = _dot(h, w1_ref[...])
    t2_ref[...] = _dot(h, w2_ref[...])
    r1_ref[...] = _dot(h, wr_ref[...]) + br_ref[...]


def _mid(r0, s1, s2, cin, cout, w1, w2, wr, br):
    nd = jax.ShapeDtypeStruct((_N, _D), jnp.float32)
    row = pl.BlockSpec((_BN, _D), lambda i: (i, 0))
    full = pl.BlockSpec((_D, _D), lambda i: (0, 0))
    bias = pl.BlockSpec((1, _D), lambda i: (0, 0))
    return pl.pallas_call(
        _mid_body,
        grid=(_N // _BN,),
        in_specs=[row, row, row, row, row, full, full, full, bias],
        out_specs=[row, row, row, row],
        out_shape=[nd, nd, nd, nd],
    )(r0, s1, s2, cin, cout, w1, w2, wr, br.reshape(1, _D))


def _fin_body(r_ref, s1_ref, s2_ref, cin_ref, cout_ref,
              x_ref, h1_ref, wfa_ref, wfb_ref, wfc_ref, bf_ref, y_ref):
    h2 = _finish(r_ref, s1_ref, s2_ref, cin_ref, cout_ref)
    y_ref[...] = (_dot(x_ref[...], wfa_ref[...])
                  + _dot(h1_ref[...], wfb_ref[...])
                  + _dot(h2, wfc_ref[...]) + bf_ref[...])


def _fin(r1, s1, s2, cin, cout, x, h1, wfa, wfb, wfc, bf):
    nd = jax.ShapeDtypeStruct((_N, _D), jnp.float32)
    row = pl.BlockSpec((_BN, _D), lambda i: (i, 0))
    full = pl.BlockSpec((_D, _D), lambda i: (0, 0))
    bias = pl.BlockSpec((1, _D), lambda i: (0, 0))
    return pl.pallas_call(
        _fin_body,
        grid=(_N // _BN,),
        in_specs=[row, row, row, row, row, row, row, full, full, full, bias],
        out_specs=row,
        out_shape=nd,
    )(r1, s1, s2, cin, cout, x, h1, wfa, wfb, wfc, bf.reshape(1, _D))


def kernel(x, edge_index, W1_0, W2_0, Wr_0, br_0,
           W1_1, W2_1, Wr_1, br_1, Wf, bf):
    src = edge_index[0]
    dst = edge_index[1]
    src2d = src.reshape(_E // _CHUNK, _CHUNK)
    dst2d = dst.reshape(_E // _CHUNK, _CHUNK)
    z128 = jnp.zeros((_RZ, _D), jnp.float32)
    ones128 = jnp.ones((_CHUNK, _D), jnp.float32)

    cnt_k = _build_cnt_kernel()
    seg_k = _build_seg_kernel()

    cin, cout = cnt_k(src2d, dst2d, ones128, z128)

    t1_0, t2_0, r0 = _mm3(x, W1_0, W2_0, Wr_0, br_0)
    s1_0, s2_0 = seg_k(t1_0, t2_0, src2d, dst2d, z128)

    h1, t1_1, t2_1, r1 = _mid(r0, s1_0, s2_0, cin, cout,
                              W1_1, W2_1, Wr_1, br_1)
    s1_1, s2_1 = seg_k(t1_1, t2_1, src2d, dst2d, z128)

    return _fin(r1, s1_1, s2_1, cin, cout, x, h1,
                Wf[:_D], Wf[_D:2 * _D], Wf[2 * _D:], bf)


# speedup vs baseline: 8.6479x; 1.0396x over previous
import functools

import jax
import jax.numpy as jnp
from jax import lax
from jax.experimental import pallas as pl
from jax.experimental.pallas import tpu as pltpu
from jax.experimental.pallas import tpu_sc as plsc

_N = 10000
_E = 320000
_D = 128

_NS = 16
_CHUNK = 125
_EPT = _E // _NS
_NCH = _EPT // _CHUNK
_BS = 32
_NB = _NCH // _BS
_NP = 10240
_RPT = _NP // _NS
_RZ = 128
_RCH = _RPT // _RZ
_CW = 16

_BN = 1000


@functools.lru_cache(maxsize=None)
def _build_seg_kernel():
    mesh = plsc.VectorSubcoreMesh(core_axis_name="c", subcore_axis_name="s")

    @functools.partial(
        pl.kernel,
        out_type=[jax.ShapeDtypeStruct((_NP, _D), jnp.float32),
                  jax.ShapeDtypeStruct((_NP, _D), jnp.float32)],
        mesh=mesh,
        scratch_types=[
            pltpu.VMEM_SHARED((_NP, _D), jnp.float32),
            pltpu.VMEM((_BS, _CHUNK), jnp.int32),
            pltpu.VMEM((_BS, _CHUNK), jnp.int32),
            pltpu.VMEM((_RZ, _D), jnp.float32),
            pltpu.VMEM((_RZ, _D), jnp.float32),
            pltpu.SemaphoreType.DMA,
            pltpu.SemaphoreType.DMA,
        ],
    )
    def seg(t1, t2, src2d, dst2d, z128,
            s1_out, s2_out, acc, gidx, sidx, rowbuf, rowbuf1, sem, sem1):
        cid = lax.axis_index("c")
        tid = lax.axis_index("s")

        def one_pass(table, g2, s3, out):
            pltpu.sync_copy(z128, rowbuf)
            for z in range(_RCH):
                pltpu.sync_copy(
                    rowbuf, acc.at[pl.ds(tid * _RPT + z * _RZ, _RZ)])
            plsc.subcore_barrier()

            bufs = (rowbuf.at[pl.ds(0, _CHUNK)], rowbuf1.at[pl.ds(0, _CHUNK)])
            sems = (sem, sem1)

            def batch(b, carry):
                r = pl.multiple_of(tid * _NCH + b * _BS, _BS)
                pltpu.sync_copy(g2.at[pl.ds(r, _BS)], gidx)
                pltpu.sync_copy(s3.at[pl.ds(r, _BS)], sidx)
                descs = [None, None]
                descs[0] = pltpu.async_copy(table.at[gidx.at[0]], bufs[0],
                                            sems[0])
                for jj in range(_BS):
                    if jj + 1 < _BS:
                        nb = (jj + 1) % 2
                        descs[nb] = pltpu.async_copy(
                            table.at[gidx.at[jj + 1]], bufs[nb], sems[nb])
                    descs[jj % 2].wait()
                    pltpu.sync_copy(bufs[jj % 2], acc.at[sidx.at[jj]],
                                    add=True)
                return carry

            lax.fori_loop(0, _NB, batch, 0)
            plsc.subcore_barrier()
            for z in range(_RCH):
                r0 = tid * _RPT + z * _RZ
                pltpu.sync_copy(acc.at[pl.ds(r0, _RZ)],
                                out.at[pl.ds(r0, _RZ)])

        @pl.when(cid == 0)
        def _():
            one_pass(t1, src2d, dst2d, s1_out)

        @pl.when(cid == 1)
        def _():
            one_pass(t2, dst2d, src2d, s2_out)

    return seg


@functools.lru_cache(maxsize=None)
def _build_cnt_kernel():
    mesh = plsc.VectorSubcoreMesh(core_axis_name="c", subcore_axis_name="s")

    @functools.partial(
        pl.kernel,
        out_type=[jax.ShapeDtypeStruct((_NP, _D), jnp.float32),
                  jax.ShapeDtypeStruct((_NP, _D), jnp.float32)],
        mesh=mesh,
        scratch_types=[
            pltpu.VMEM_SHARED((_NP, _D), jnp.float32),
            pltpu.VMEM((_BS, _CHUNK), jnp.int32),
            pltpu.VMEM((_CHUNK, _D), jnp.float32),
            pltpu.VMEM((_RZ, _D), jnp.float32),
        ],
    )
    def cnt(src2d, dst2d, ones128, z128,
            cin_out, cout_out, acc, sidx, onesbuf, rowbuf):
        cid = lax.axis_index("c")
        tid = lax.axis_index("s")

        def one_pass(s3, out):
            pltpu.sync_copy(z128, rowbuf)
            for z in range(_RCH):
                pltpu.sync_copy(
                    rowbuf, acc.at[pl.ds(tid * _RPT + z * _RZ, _RZ)])
            pltpu.sync_copy(ones128, onesbuf)
            plsc.subcore_barrier()

            def batch(b, carry):
                r = pl.multiple_of(tid * _NCH + b * _BS, _BS)
                pltpu.sync_copy(s3.at[pl.ds(r, _BS)], sidx)
                for jj in range(_BS):
                    pltpu.sync_copy(onesbuf, acc.at[sidx.at[jj]], add=True)
                return carry

            lax.fori_loop(0, _NB, batch, 0)
            plsc.subcore_barrier()
            for z in range(_RCH):
                r0 = tid * _RPT + z * _RZ
                pltpu.sync_copy(acc.at[pl.ds(r0, _RZ)],
                                out.at[pl.ds(r0, _RZ)])

        @pl.when(cid == 0)
        def _():
            one_pass(dst2d, cin_out)

        @pl.when(cid == 1)
        def _():
            one_pass(src2d, cout_out)

    return cnt


def _dot(a, b):
    return jnp.dot(a, b, preferred_element_type=jnp.float32)


def _mm3_body(x_ref, w1_ref, w2_ref, wr_ref, br_ref, t1_ref, t2_ref, r_ref):
    xb = x_ref[...]
    t1_ref[...] = _dot(xb, w1_ref[...])
    t2_ref[...] = _dot(xb, w2_ref[...])
    r_ref[...] = _dot(xb, wr_ref[...]) + br_ref[...]


def _mm3(x, w1, w2, wr, br):
    nd = jax.ShapeDtypeStruct((_N, _D), jnp.float32)
    row = pl.BlockSpec((_BN, _D), lambda i: (i, 0))
    full = pl.BlockSpec((_D, _D), lambda i: (0, 0))
    bias = pl.BlockSpec((1, _D), lambda i: (0, 0))
    return pl.pallas_call(
        _mm3_body,
        grid=(_N // _BN,),
        in_specs=[row, full, full, full, bias],
        out_specs=[row, row, row],
        out_shape=[nd, nd, nd],
    )(x, w1, w2, wr, br.reshape(1, _D))


def _finish(r_ref, s1_ref, s2_ref, cin_ref, cout_ref):
    rin = 1.0 / jnp.maximum(cin_ref[...], 1.0)
    rout = 1.0 / jnp.maximum(cout_ref[...], 1.0)
    return jnp.maximum(
        r_ref[...] + s1_ref[...] * rin + s2_ref[...] * rout, 0.0)


def _mid_body(r_ref, s1_ref, s2_ref, cin_ref, cout_ref,
              w1_ref, w2_ref, wr_ref, br_ref,
              h_ref, t1_ref, t2_ref, r1_ref):
    h = _finish(r_ref, s1_ref, s2_ref, cin_ref, cout_ref)
    h_ref[...] = h
    t1_ref[...] = _dot(h, w1_ref[...])
    t2_ref[...] = _dot(h, w2_ref[...])
    r1_ref[...] = _dot(h, wr_ref[...]) + br_ref[...]


def _mid(r0, s1, s2, cin, cout, w1, w2, wr, br):
    nd = jax.ShapeDtypeStruct((_N, _D), jnp.float32)
    row = pl.BlockSpec((_BN, _D), lambda i: (i, 0))
    full = pl.BlockSpec((_D, _D), lambda i: (0, 0))
    bias = pl.BlockSpec((1, _D), lambda i: (0, 0))
    return pl.pallas_call(
        _mid_body,
        grid=(_N // _BN,),
        in_specs=[row, row, row, row, row, full, full, full, bias],
        out_specs=[row, row, row, row],
        out_shape=[nd, nd, nd, nd],
    )(r0, s1, s2, cin, cout, w1, w2, wr, br.reshape(1, _D))


def _fin_body(r_ref, s1_ref, s2_ref, cin_ref, cout_ref,
              x_ref, h1_ref, wfa_ref, wfb_ref, wfc_ref, bf_ref, y_ref):
    h2 = _finish(r_ref, s1_ref, s2_ref, cin_ref, cout_ref)
    y_ref[...] = (_dot(x_ref[...], wfa_ref[...])
                  + _dot(h1_ref[...], wfb_ref[...])
                  + _dot(h2, wfc_ref[...]) + bf_ref[...])


def _fin(r1, s1, s2, cin, cout, x, h1, wfa, wfb, wfc, bf):
    nd = jax.ShapeDtypeStruct((_N, _D), jnp.float32)
    row = pl.BlockSpec((_BN, _D), lambda i: (i, 0))
    full = pl.BlockSpec((_D, _D), lambda i: (0, 0))
    bias = pl.BlockSpec((1, _D), lambda i: (0, 0))
    return pl.pallas_call(
        _fin_body,
        grid=(_N // _BN,),
        in_specs=[row, row, row, row, row, row, row, full, full, full, bias],
        out_specs=row,
        out_shape=nd,
    )(r1, s1, s2, cin, cout, x, h1, wfa, wfb, wfc, bf.reshape(1, _D))


def kernel(x, edge_index, W1_0, W2_0, Wr_0, br_0,
           W1_1, W2_1, Wr_1, br_1, Wf, bf):
    src = edge_index[0]
    dst = edge_index[1]
    src2d = src.reshape(_E // _CHUNK, _CHUNK)
    dst2d = dst.reshape(_E // _CHUNK, _CHUNK)
    z128 = jnp.zeros((_RZ, _D), jnp.float32)
    ones128 = jnp.ones((_CHUNK, _D), jnp.float32)

    cnt_k = _build_cnt_kernel()
    seg_k = _build_seg_kernel()

    cin, cout = cnt_k(src2d, dst2d, ones128, z128)

    t1_0, t2_0, r0 = _mm3(x, W1_0, W2_0, Wr_0, br_0)
    s1_0, s2_0 = seg_k(t1_0, t2_0, src2d, dst2d, z128)

    h1, t1_1, t2_1, r1 = _mid(r0, s1_0, s2_0, cin, cout,
                              W1_1, W2_1, Wr_1, br_1)
    s1_1, s2_1 = seg_k(t1_1, t2_1, src2d, dst2d, z128)

    return _fin(r1, s1_1, s2_1, cin, cout, x, h1,
                Wf[:_D], Wf[_D:2 * _D], Wf[2 * _D:], bf)
